# Initial kernel scaffold; baseline (speedup 1.0000x reference)
import jax
import jax.numpy as jnp
from jax.experimental import pallas as pl


def kernel(x, edge_index, W_in, b_in, gcn1_W, gcn1_b, gcn2_W, gcn2_b, gat_W, gat_a_src, gat_a_dst, gat_b, W_out, b_out):
    raise NotImplementedError("write your pallas kernel here")


# speedup vs baseline: 11.8201x; 11.8201x over previous
import functools

import jax
import jax.numpy as jnp
from jax import lax
from jax.experimental import pallas as pl
from jax.experimental.pallas import tpu as pltpu
from jax.experimental.pallas import tpu_sc as plsc

N = 50000
NPAD = 51200
E = 800000
EPAD = 802816
PADE = EPAD - E
IN_DIM = 128
EMB = 64
HEADS = 4
ER = EPAD // 128

NC, NS = 2, 16
HALF = 25600
TRASH = 512
ACC = HALF + TRASH
QHALF = 12800
ACCQ = QHALF + TRASH

f32 = jnp.float32
i32 = jnp.int32

_mesh = plsc.VectorSubcoreMesh(core_axis_name="c", subcore_axis_name="s")


def _zero_vmem(buf, nrows, ncols):
    z = jnp.zeros((16,), f32)
    cpr = ncols // 16

    def body(v, _):
        buf[v // cpr, pl.ds((v % cpr) * 16, 16)] = z
        return 0

    lax.fori_loop(0, nrows * cpr, body, 0)


def _sc_degree(dst_rows):
    @functools.partial(
        pl.kernel,
        out_type=jax.ShapeDtypeStruct((NC, NPAD), f32),
        mesh=_mesh,
        compiler_params=pltpu.CompilerParams(use_tc_tiling_on_sc=False),
        scratch_types=[
            pltpu.VMEM((4, 128), f32),
            pltpu.VMEM((4, 128), i32),
            pltpu.VMEM((3200,), f32),
            pltpu.VMEM_SHARED((NPAD,), f32),
        ],
    )
    def k(dst_hbm, out_hbm, ones, dbuf, zbuf, acc):
        c = lax.axis_index("c")
        s = lax.axis_index("s")
        w = s * NC + c

        z = jnp.zeros((16,), f32)

        def zero(v, _):
            zbuf[pl.ds(v * 16, 16)] = z
            return 0

        lax.fori_loop(0, 200, zero, 0)
        one = jnp.ones((16,), f32)
        for r in range(4):
            for t in range(8):
                ones[r, pl.ds(t * 16, 16)] = one
        pltpu.sync_copy(zbuf, acc.at[pl.ds(s * 3200, 3200)])
        plsc.subcore_barrier()

        def chunk(i, _):
            row0 = (w * 49 + i) * 4
            pltpu.sync_copy(dst_hbm.at[pl.ds(row0, 4)], dbuf)
            for r in range(4):
                pltpu.sync_copy(ones.at[r], acc.at[dbuf.at[r]], add=True)
            return 0

        lax.fori_loop(0, 49, chunk, 0)
        plsc.subcore_barrier()
        pltpu.sync_copy(acc.at[pl.ds(s * 3200, 3200)],
                        out_hbm.at[c].at[pl.ds(s * 3200, 3200)])

    return k(dst_rows)


def _sc_scatter_rows(table, src_rows, dst_rows, base):
    @functools.partial(
        pl.kernel,
        out_type=jax.ShapeDtypeStruct((NC * QHALF, EMB), f32),
        mesh=_mesh,
        compiler_params=pltpu.CompilerParams(use_tc_tiling_on_sc=False),
        scratch_types=[
            pltpu.VMEM((4, 128), i32),
            pltpu.VMEM((4, 128), i32),
            pltpu.VMEM((4, 128), i32),
            pltpu.VMEM((512, EMB), f32),
            pltpu.VMEM((416, EMB), f32),
            pltpu.VMEM_SHARED((ACCQ, EMB), f32),
            pltpu.SemaphoreType.DMA,
        ],
    )
    def k(t_hbm, src_hbm, dst_hbm, s_hbm, sbuf, dbuf, dloc, rows, zbuf,
          acc, sem):
        c = lax.axis_index("c")
        s = lax.axis_index("s")
        lo = base + c * QHALF

        _zero_vmem(zbuf, 416, EMB)
        for j in range(2):
            pltpu.sync_copy(zbuf, acc.at[pl.ds(s * 832 + j * 416, 416)])
        plsc.subcore_barrier()

        def chunk(i, _):
            row0 = (s * 98 + i) * 4
            pltpu.sync_copy(src_hbm.at[pl.ds(row0, 4)], sbuf)
            pltpu.sync_copy(dst_hbm.at[pl.ds(row0, 4)], dbuf)
            for r in range(4):
                for t in range(8):
                    d = dbuf[r, pl.ds(t * 16, 16)]
                    m = (d >= lo) & (d < lo + QHALF)
                    dloc[r, pl.ds(t * 16, 16)] = jnp.where(
                        m, d - lo, QHALF + (d & (TRASH - 1)))
            descs = [pltpu.async_copy(t_hbm.at[sbuf.at[r]],
                                      rows.at[pl.ds(r * 128, 128)], sem)
                     for r in range(4)]
            for dsc in descs:
                dsc.wait()
            for r in range(4):
                pltpu.sync_copy(rows.at[pl.ds(r * 128, 128)],
                                acc.at[dloc.at[r]], add=True)
            return 0

        lax.fori_loop(0, 98, chunk, 0)
        plsc.subcore_barrier()
        for j in range(2):
            r0 = s * 800 + j * 400
            pltpu.sync_copy(acc.at[pl.ds(r0, 400)],
                            s_hbm.at[pl.ds(c * QHALF + r0, 400)])

    return k(table, src_rows, dst_rows)


def _sc_gat_den(asrc16, adst16, src_rows, dst_rows):
    @functools.partial(
        pl.kernel,
        out_type=jax.ShapeDtypeStruct((NPAD, 16), f32),
        mesh=_mesh,
        compiler_params=pltpu.CompilerParams(use_tc_tiling_on_sc=False),
        scratch_types=[
            pltpu.VMEM((4, 128), i32),
            pltpu.VMEM((4, 128), i32),
            pltpu.VMEM((4, 128), i32),
            pltpu.VMEM((512, 16), f32),
            pltpu.VMEM((512, 16), f32),
            pltpu.VMEM((512, 16), f32),
            pltpu.VMEM((408, 16), f32),
            pltpu.VMEM_SHARED((ACC, 16), f32),
            pltpu.SemaphoreType.DMA,
        ],
    )
    def k(a_hbm, ad_hbm, src_hbm, dst_hbm, den_hbm, sbuf, dbuf, dloc,
          arows, drows, exrows, zbuf, acc, sem):
        c = lax.axis_index("c")
        s = lax.axis_index("s")
        lo = c * HALF

        _zero_vmem(zbuf, 408, 16)
        for j in range(4):
            pltpu.sync_copy(zbuf, acc.at[pl.ds(s * 1632 + j * 408, 408)])
        plsc.subcore_barrier()

        def chunk(i, _):
            row0 = (s * 98 + i) * 4
            pltpu.sync_copy(src_hbm.at[pl.ds(row0, 4)], sbuf)
            pltpu.sync_copy(dst_hbm.at[pl.ds(row0, 4)], dbuf)
            for r in range(4):
                for t in range(8):
                    d = dbuf[r, pl.ds(t * 16, 16)]
                    m = (d >= lo) & (d < lo + HALF)
                    dloc[r, pl.ds(t * 16, 16)] = jnp.where(
                        m, d - lo, HALF + (d & (TRASH - 1)))
            descs = [pltpu.async_copy(a_hbm.at[sbuf.at[r]],
                                      arows.at[pl.ds(r * 128, 128)], sem)
                     for r in range(4)]
            descs += [pltpu.async_copy(ad_hbm.at[dbuf.at[r]],
                                       drows.at[pl.ds(r * 128, 128)], sem)
                      for r in range(4)]
            for dsc in descs:
                dsc.wait()

            def edge(e, _):
                for u in range(8):
                    a = arows[e * 8 + u] + drows[e * 8 + u]
                    l = jnp.where(a > 0, a, 0.2 * a)
                    exrows[e * 8 + u] = jnp.exp(l)
                return 0

            lax.fori_loop(0, 64, edge, 0)
            for r in range(4):
                pltpu.sync_copy(exrows.at[pl.ds(r * 128, 128)],
                                acc.at[dloc.at[r]], add=True)
            return 0

        lax.fori_loop(0, 98, chunk, 0)
        plsc.subcore_barrier()
        for j in range(4):
            r0 = s * 1600 + j * 400
            pltpu.sync_copy(acc.at[pl.ds(r0, 400)],
                            den_hbm.at[pl.ds(lo + r0, 400)])

    return k(asrc16, adst16, src_rows, dst_rows)


def _sc_gat_agg(hw4, asrc16, dt16, src_rows, dst_rows, base):

    @functools.partial(
        pl.kernel,
        out_type=jax.ShapeDtypeStruct((NC * QHALF, EMB), f32),
        mesh=_mesh,
        compiler_params=pltpu.CompilerParams(use_tc_tiling_on_sc=False),
        scratch_types=[
            pltpu.VMEM((1, 128), i32),
            pltpu.VMEM((1, 128), i32),
            pltpu.VMEM((1, 128), i32),
            pltpu.VMEM((128, 4 * EMB), f32),
            pltpu.VMEM((128, 16), f32),
            pltpu.VMEM((128, 16), f32),
            pltpu.VMEM((128, EMB), f32),
            pltpu.VMEM((416, EMB), f32),
            pltpu.VMEM_SHARED((ACCQ, EMB), f32),
            pltpu.SemaphoreType.DMA,
        ],
    )
    def k(h_hbm, a_hbm, dt_hbm, src_hbm, dst_hbm, g_hbm, sbuf, dbuf, dloc,
          rv, arows, dtrows, msg, zbuf, acc, sem):
        c = lax.axis_index("c")
        s = lax.axis_index("s")
        lo = base + c * QHALF
        islog = lax.iota(i32, 16) < 4

        _zero_vmem(zbuf, 416, EMB)
        for j in range(2):
            pltpu.sync_copy(zbuf, acc.at[pl.ds(s * 832 + j * 416, 416)])
        plsc.subcore_barrier()

        def chunk(i, _):
            row0 = s * 392 + i
            pltpu.sync_copy(src_hbm.at[pl.ds(row0, 1)], sbuf)
            pltpu.sync_copy(dst_hbm.at[pl.ds(row0, 1)], dbuf)
            for t in range(8):
                d = dbuf[0, pl.ds(t * 16, 16)]
                m = (d >= lo) & (d < lo + QHALF)
                dloc[0, pl.ds(t * 16, 16)] = jnp.where(
                    m, d - lo, QHALF + (d & (TRASH - 1)))
            descs = [
                pltpu.async_copy(h_hbm.at[sbuf.at[0]], rv, sem),
                pltpu.async_copy(a_hbm.at[sbuf.at[0]], arows, sem),
                pltpu.async_copy(dt_hbm.at[dbuf.at[0]], dtrows, sem),
            ]
            for dsc in descs:
                dsc.wait()

            def edgem(e, _):
                x = arows[e] + dtrows[e]
                l = jnp.where(x > 0, x, 0.2 * x)
                wv = jnp.where(islog, jnp.exp(l), l)
                w0 = wv[0] * wv[4]
                w1 = wv[1] * wv[5]
                w2 = wv[2] * wv[6]
                w3 = wv[3] * wv[7]
                for q in range(4):
                    v = w0 * rv[e, pl.ds(q * 16, 16)]
                    v += w1 * rv[e, pl.ds(64 + q * 16, 16)]
                    v += w2 * rv[e, pl.ds(128 + q * 16, 16)]
                    v += w3 * rv[e, pl.ds(192 + q * 16, 16)]
                    msg[e, pl.ds(q * 16, 16)] = v
                return 0

            lax.fori_loop(0, 128, edgem, 0)
            pltpu.sync_copy(msg, acc.at[dloc.at[0]], add=True)
            return 0

        lax.fori_loop(0, 392, chunk, 0)
        plsc.subcore_barrier()
        for j in range(2):
            r0 = s * 800 + j * 400
            pltpu.sync_copy(acc.at[pl.ds(r0, 400)],
                            g_hbm.at[pl.ds(c * QHALF + r0, 400)])

    return k(hw4, asrc16, dt16, src_rows, dst_rows)


_R = 2048
_GRID = NPAD // _R


def _rowspec(cols):
    return pl.BlockSpec((_R, cols), lambda i: (i, 0))


def _fullspec(a, b):
    return pl.BlockSpec((a, b), lambda i: (0, 0))


def _degspec():
    return pl.BlockSpec((NC, _R), lambda i: (0, i))


def _dinv(degp):
    return lax.rsqrt(degp[0] + degp[1] + 1.0)


def _tc_in(x, w, b):
    def body(x_ref, w_ref, b_ref, o_ref):
        o_ref[...] = jnp.maximum(x_ref[...] @ w_ref[...] + b_ref[...], 0.0)

    return pl.pallas_call(
        body,
        grid=(_GRID,),
        in_specs=[_rowspec(IN_DIM), _fullspec(IN_DIM, EMB), _fullspec(1, EMB)],
        out_specs=_rowspec(EMB),
        out_shape=jax.ShapeDtypeStruct((NPAD, EMB), f32),
    )(x, w, b.reshape(1, EMB))


def _tc_t1(h0, w, degp):
    def body(h_ref, w_ref, g_ref, o_ref):
        dinv = _dinv(g_ref[...])
        o_ref[...] = dinv[:, None] * (h_ref[...] @ w_ref[...])

    return pl.pallas_call(
        body,
        grid=(_GRID,),
        in_specs=[_rowspec(EMB), _fullspec(EMB, EMB), _degspec()],
        out_specs=_rowspec(EMB),
        out_shape=jax.ShapeDtypeStruct((NPAD, EMB), f32),
    )(h0, w, degp)


def _tc_mid(S, t, degp, b, w2):
    def body(s_ref, t_ref, g_ref, b_ref, w_ref, o_ref):
        dinv = _dinv(g_ref[...])
        h = jnp.maximum(dinv[:, None] * (s_ref[...] + t_ref[...])
                        + b_ref[...], 0.0)
        o_ref[...] = dinv[:, None] * (h @ w_ref[...])

    return pl.pallas_call(
        body,
        grid=(_GRID,),
        in_specs=[_rowspec(EMB), _rowspec(EMB), _degspec(),
                  _fullspec(1, EMB), _fullspec(EMB, EMB)],
        out_specs=_rowspec(EMB),
        out_shape=jax.ShapeDtypeStruct((NPAD, EMB), f32),
    )(S, t, degp, b.reshape(1, EMB), w2)


def _tc_gatpre(S, t, degp, b, gw, asel, dsel):
    def body(s_ref, t_ref, g_ref, b_ref, w_ref, a_ref, d_ref,
             hw_ref, as_ref, ad_ref):
        dinv = _dinv(g_ref[...])
        h = jnp.maximum(dinv[:, None] * (s_ref[...] + t_ref[...])
                        + b_ref[...], 0.0)
        hw = h @ w_ref[...]
        hw_ref[...] = hw
        as_ref[...] = hw @ a_ref[...]
        ad_ref[...] = hw @ d_ref[...]

    return pl.pallas_call(
        body,
        grid=(_GRID,),
        in_specs=[_rowspec(EMB), _rowspec(EMB), _degspec(),
                  _fullspec(1, EMB), _fullspec(EMB, HEADS * EMB),
                  _fullspec(HEADS * EMB, 16), _fullspec(HEADS * EMB, 16)],
        out_specs=(_rowspec(HEADS * EMB), _rowspec(16), _rowspec(16)),
        out_shape=(jax.ShapeDtypeStruct((NPAD, HEADS * EMB), f32),
                   jax.ShapeDtypeStruct((NPAD, 16), f32),
                   jax.ShapeDtypeStruct((NPAD, 16), f32)),
    )(S, t, degp, b.reshape(1, EMB), gw, asel, dsel)


def _tc_gatmid(asrc16, adst16, den):
    def body(a_ref, d_ref, n_ref, o_ref):
        a = a_ref[...]
        d = d_ref[...]
        x = a + d
        l = jnp.where(x > 0, x, 0.2 * x)
        exs = jnp.exp(l)
        r = 0.25 / (n_ref[...] + exs)
        lanes = lax.broadcasted_iota(i32, (_R, 16), 1)
        o_ref[...] = jnp.where(lanes < 4, d,
                               jnp.where(lanes < 8,
                                         jnp.roll(r, 4, axis=1), 0.0))

    return pl.pallas_call(
        body,
        grid=(_GRID,),
        in_specs=[_rowspec(16), _rowspec(16), _rowspec(16)],
        out_specs=_rowspec(16),
        out_shape=jax.ShapeDtypeStruct((NPAD, 16), f32),
    )(asrc16, adst16, den)


def _tc_final(G, hw4, dt16, asrc16, adst16, gb, wout, bout):
    def body(g_ref, h_ref, dt_ref, a_ref, d_ref, gb_ref, w_ref, bo_ref,
             o_ref):
        x = a_ref[...] + d_ref[...]
        l = jnp.where(x > 0, x, 0.2 * x)
        exs = jnp.exp(l)
        r = jnp.roll(dt_ref[...], -4, axis=1)
        wself = exs * r
        gt = g_ref[...]
        hw = h_ref[...]
        for h in range(HEADS):
            gt = gt + wself[:, h:h + 1] * hw[:, h * EMB:(h + 1) * EMB]
        o_ref[...] = (gt + gb_ref[...]) @ w_ref[...] + bo_ref[...]

    return pl.pallas_call(
        body,
        grid=(_GRID,),
        in_specs=[_rowspec(EMB), _rowspec(HEADS * EMB), _rowspec(16),
                  _rowspec(16), _rowspec(16), _fullspec(1, EMB),
                  _fullspec(EMB, EMB), _fullspec(1, EMB)],
        out_specs=_rowspec(EMB),
        out_shape=jax.ShapeDtypeStruct((NPAD, EMB), f32),
    )(G, hw4, dt16, asrc16, adst16, gb.reshape(1, EMB), wout,
      bout.reshape(1, EMB))


def kernel(x, edge_index, W_in, b_in, gcn1_W, gcn1_b, gcn2_W, gcn2_b,
           gat_W, gat_a_src, gat_a_dst, gat_b, W_out, b_out):
    ei = edge_index.astype(i32)
    padi = jnp.arange(PADE, dtype=i32)
    src = jnp.concatenate([ei[0], padi & 1023]).reshape(ER, 128)
    dst = jnp.concatenate([ei[1], N + (padi & 511)]).reshape(ER, 128)
    xp = jnp.pad(x, ((0, NPAD - N), (0, 0)))

    sel = jnp.eye(HEADS, 16, dtype=f32)
    asel = (gat_a_src[:, :, None] * sel[:, None, :]).reshape(HEADS * EMB, 16)
    dsel = (gat_a_dst[:, :, None] * sel[:, None, :]).reshape(HEADS * EMB, 16)

    degp = _sc_degree(dst).reshape(NC, NPAD)
    h0 = _tc_in(xp, W_in, b_in)
    t1 = _tc_t1(h0, gcn1_W, degp)
    S1 = jnp.concatenate([_sc_scatter_rows(t1, src, dst, 0),
                          _sc_scatter_rows(t1, src, dst, NPAD // 2)])
    t2 = _tc_mid(S1, t1, degp, gcn1_b, gcn2_W)
    S2 = jnp.concatenate([_sc_scatter_rows(t2, src, dst, 0),
                          _sc_scatter_rows(t2, src, dst, NPAD // 2)])
    hw4, asrc16, adst16 = _tc_gatpre(S2, t2, degp, gcn2_b, gat_W, asel, dsel)
    den = _sc_gat_den(asrc16, adst16, src, dst)
    dt16 = _tc_gatmid(asrc16, adst16, den)
    G = jnp.concatenate([_sc_gat_agg(hw4, asrc16, dt16, src, dst, 0),
                         _sc_gat_agg(hw4, asrc16, dt16, src, dst, NPAD // 2)])
    out = _tc_final(G, hw4, dt16, asrc16, adst16, gat_b, W_out, b_out)
    return out[:N]
